# baseline (speedup 1.0000x reference)
import functools

import jax
import jax.numpy as jnp
from jax import lax
from jax.experimental import pallas as pl
from jax.experimental.pallas import tpu as pltpu
from jax.experimental.pallas import tpu_sc as plsc

N_NODES = 10000
N_EDGES = 160000
NC = 2
NS = 16
BATCH = 128
EPAD = 163840
NB_AGG = EPAD // (NS * BATCH)
NB_HALF = NB_AGG // 2
NB_DEG = EPAD // (NC * NS * BATCH)
RPT = 632
NACC = NS * RPT
DUMP_ROW = N_NODES
RB = 1000
RB2 = 632


def _zero_vmem_rows(ref, nrows, ncols):
    zero16 = jnp.zeros((16,), jnp.float32)

    def row(i, carry):
        for k in range(ncols // 16):
            ref[i, pl.ds(k * 16, 16)] = zero16
        return carry

    lax.fori_loop(0, nrows, row, 0)


def _deg_body(dst_hbm, out_hbm, didx, onesb, zbuf, acc):
    c = lax.axis_index("c")
    s = lax.axis_index("s")
    w = c * NS + s

    one16 = jnp.ones((16,), jnp.float32)

    def fill(i, carry):
        for k in range(128 // 16):
            onesb[i, pl.ds(k * 16, 16)] = one16
        return carry

    lax.fori_loop(0, BATCH, fill, 0)
    _zero_vmem_rows(zbuf, BATCH, 128)
    for t in range(-(-RPT // BATCH)):
        rows = min(BATCH, RPT - t * BATCH)
        pltpu.sync_copy(zbuf.at[pl.ds(0, rows)],
                        acc.at[pl.ds(s * RPT + t * BATCH, rows)])
    plsc.subcore_barrier()

    pltpu.sync_copy(dst_hbm.at[w], didx)

    def ebody(j, carry):
        pltpu.sync_copy(onesb, acc.at[didx.at[j]], add=True)
        return carry

    lax.fori_loop(0, NB_DEG, ebody, 0)
    plsc.subcore_barrier()
    for t in range(-(-RPT // BATCH)):
        rows = min(BATCH, RPT - t * BATCH)
        pltpu.sync_copy(acc.at[pl.ds(s * RPT + t * BATCH, rows)],
                        zbuf.at[pl.ds(0, rows)])
        pltpu.sync_copy(zbuf.at[pl.ds(0, rows)],
                        out_hbm.at[pl.ds(c * NACC + s * RPT + t * BATCH, rows)])


def _sc_deg(dst_deg):
    mesh = plsc.VectorSubcoreMesh(core_axis_name="c", subcore_axis_name="s")
    return pl.kernel(
        _deg_body,
        out_type=jax.ShapeDtypeStruct((NC * NACC, 128), jnp.float32),
        mesh=mesh,
        scratch_types=[
            pltpu.VMEM((NB_DEG, BATCH), jnp.int32),
            pltpu.VMEM((BATCH, 128), jnp.float32),
            pltpu.VMEM((BATCH, 128), jnp.float32),
            pltpu.VMEM_SHARED((NACC, 128), jnp.float32),
        ],
    )(dst_deg)


def _agg_run_slot(ck, h0, h1, slot, fc, u_hbm, srcoff_hbm, dst_hbm, out_hbm,
                  sidx, didx, gbufa, gbufb, sema, semb, ssema, ssemb, acc, s):
    _zero_vmem_rows(gbufa, BATCH, fc)
    zcps = []
    for t in range(-(-RPT // BATCH)):
        rows = min(BATCH, RPT - t * BATCH)
        zcps.append(pltpu.async_copy(
            gbufa.at[pl.ds(0, rows)],
            acc.at[pl.ds(s * RPT + t * BATCH, rows)], sema))
    for cp in zcps:
        cp.wait()
    plsc.subcore_barrier()

    def g_start(j, buf, sem):
        pltpu.async_copy(u_hbm.at[sidx.at[j]], buf, sem)

    def g_wait(j, buf, sem):
        pltpu.make_async_copy(u_hbm.at[sidx.at[j]], buf, sem).wait()

    def s_start(j, buf, sem):
        pltpu.async_copy(buf, acc.at[didx.at[j]], sem, add=True)

    def s_wait(j, buf, sem):
        pltpu.make_async_copy(buf, acc.at[didx.at[j]], sem).wait()

    for h in range(h0, h1):
        cps = [pltpu.async_copy(srcoff_hbm.at[(ck * NS + s) * 2 + h], sidx, sema),
               pltpu.async_copy(dst_hbm.at[s * 2 + h], didx, semb)]
        for cp in cps:
            cp.wait()

        g_start(0, gbufa, sema)
        g_wait(0, gbufa, sema)
        s_start(0, gbufa, ssema)
        g_start(1, gbufb, semb)

        def pair(i, carry):
            j1 = 2 * i + 1
            g_wait(j1, gbufb, semb)
            s_start(j1, gbufb, ssemb)
            s_wait(j1 - 1, gbufa, ssema)
            g_start(j1 + 1, gbufa, sema)
            j2 = j1 + 1
            g_wait(j2, gbufa, sema)
            s_start(j2, gbufa, ssema)
            s_wait(j2 - 1, gbufb, ssemb)
            g_start(j2 + 1, gbufb, semb)
            return carry

        lax.fori_loop(0, NB_HALF // 2 - 1, pair, 0)
        jl = NB_HALF - 1
        g_wait(jl, gbufb, semb)
        s_start(jl, gbufb, ssemb)
        s_wait(jl - 1, gbufa, ssema)
        s_wait(jl, gbufb, ssemb)
    plsc.subcore_barrier()
    stages = [gbufa, gbufb]
    wcps = {}
    for t in range(-(-RPT // BATCH)):
        rows = min(BATCH, RPT - t * BATCH)
        stg = stages[t % 2]
        if t >= 2:
            wcps[t - 2].wait()
        pltpu.async_copy(acc.at[pl.ds(s * RPT + t * BATCH, rows)],
                         stg.at[pl.ds(0, rows)], sema).wait()
        wcps[t] = pltpu.async_copy(
            stg.at[pl.ds(0, rows)],
            out_hbm.at[pl.ds(slot * NACC + s * RPT + t * BATCH, rows)], semb)
    for t in sorted(wcps)[-2:]:
        wcps[t].wait()


def _agg_body(sched0, sched1, fc, u_hbm, srcoff_hbm, dst_hbm, out_hbm,
              sidx, didx, gbufa, gbufb, sema, semb, ssema, ssemb, acc):
    c = lax.axis_index("c")
    s = lax.axis_index("s")
    args = (fc, u_hbm, srcoff_hbm, dst_hbm, out_hbm,
            sidx, didx, gbufa, gbufb, sema, semb, ssema, ssemb, acc, s)

    @pl.when(c == 0)
    def _core0():
        for ck, h0, h1, slot in sched0:
            _agg_run_slot(ck, h0, h1, slot, *args)

    @pl.when(c == 1)
    def _core1():
        for ck, h0, h1, slot in sched1:
            _agg_run_slot(ck, h0, h1, slot, *args)


def _sc_agg(u_flat, srcoff, dst_agg, sched0, sched1, nslots, fc):
    mesh = plsc.VectorSubcoreMesh(core_axis_name="c", subcore_axis_name="s")
    out = pl.kernel(
        functools.partial(_agg_body, sched0, sched1, fc),
        out_type=jax.ShapeDtypeStruct((nslots * NACC, fc), jnp.float32),
        mesh=mesh,
        scratch_types=[
            pltpu.VMEM((NB_HALF, BATCH), jnp.int32),
            pltpu.VMEM((NB_HALF, BATCH), jnp.int32),
            pltpu.VMEM((BATCH, fc), jnp.float32),
            pltpu.VMEM((BATCH, fc), jnp.float32),
            pltpu.SemaphoreType.DMA,
            pltpu.SemaphoreType.DMA,
            pltpu.SemaphoreType.DMA,
            pltpu.SemaphoreType.DMA,
            pltpu.VMEM_SHARED((NACC, fc), jnp.float32),
        ],
    )(u_flat, srcoff, dst_agg)
    return out.reshape(nslots, NACC, fc)


def _tc1_body(x_ref, w_ref, deg_ref, out_ref):
    y = jnp.dot(x_ref[...], w_ref[...], preferred_element_type=jnp.float32)
    out_ref[0] = y * lax.rsqrt(deg_ref[...])


def _tc1(x, W1, deg):
    return pl.pallas_call(
        _tc1_body,
        grid=(4, N_NODES // RB),
        in_specs=[
            pl.BlockSpec((RB, 256), lambda co, r: (r, 0)),
            pl.BlockSpec((256, 128), lambda co, r: (0, co)),
            pl.BlockSpec((RB, 1), lambda co, r: (r, 0)),
        ],
        out_specs=pl.BlockSpec((1, RB, 128), lambda co, r: (co, r, 0)),
        out_shape=jax.ShapeDtypeStruct((4, NACC, 128), jnp.float32),
        compiler_params=pltpu.CompilerParams(
            dimension_semantics=("parallel", "parallel")),
    )(x, W1, deg)


def _tc2_body(nci, agg_ref, u_ref, b_ref, deg_ref, w_ref, dego_ref, bo_ref, out_ref):
    ci = pl.program_id(2)
    h = (agg_ref[0] + u_ref[0]) * lax.rsqrt(deg_ref[...]) + b_ref[0]
    h = jnp.where(h >= 0, h, 0.01 * h)
    part = jnp.dot(h, w_ref[0, 0], preferred_element_type=jnp.float32)

    @pl.when(ci == 0)
    def _init():
        out_ref[0] = part

    @pl.when(ci > 0)
    def _acc():
        out_ref[0] += part

    @pl.when(ci == nci - 1)
    def _fin():
        out_ref[0] = out_ref[0] * lax.rsqrt(dego_ref[...]) + bo_ref[0]


def _tc2(agg, u, bl, deg, W, dego, bo, nci, fci, nco, fco):
    return pl.pallas_call(
        functools.partial(_tc2_body, nci),
        grid=(nco, NACC // RB2, nci),
        in_specs=[
            pl.BlockSpec((1, RB2, fci), lambda co, r, ci: (ci, r, 0)),
            pl.BlockSpec((1, RB2, fci), lambda co, r, ci: (ci, r, 0)),
            pl.BlockSpec((1, 1, fci), lambda co, r, ci: (ci, 0, 0)),
            pl.BlockSpec((RB2, 1), lambda co, r, ci: (r, 0)),
            pl.BlockSpec((1, 1, fci, fco), lambda co, r, ci: (ci, co, 0, 0)),
            pl.BlockSpec((RB2, 1), lambda co, r, ci: (r, 0)),
            pl.BlockSpec((1, 1, fco), lambda co, r, ci: (co, 0, 0)),
        ],
        out_specs=pl.BlockSpec((1, RB2, fco), lambda co, r, ci: (co, r, 0)),
        out_shape=jax.ShapeDtypeStruct((nco, NACC, fco), jnp.float32),
        compiler_params=pltpu.CompilerParams(
            dimension_semantics=("parallel", "parallel", "arbitrary")),
    )(agg, u, bl, deg, W, dego, bo)


def kernel(x, edge_index, W1, b1, W2, b2, W3, b3, Wc, bc):
    src = edge_index[0].astype(jnp.int32)
    dst = edge_index[1].astype(jnp.int32)
    pad = EPAD - N_EDGES
    src_p = jnp.concatenate([src, jnp.zeros((pad,), jnp.int32)])
    dst_p = jnp.concatenate([dst, jnp.full((pad,), DUMP_ROW, jnp.int32)])
    dst_agg = dst_p.reshape(NS * 2, NB_HALF, BATCH)
    dst_deg = dst_p.reshape(NC * NS, NB_DEG, BATCH)
    chunk_off = (jnp.arange(4, dtype=jnp.int32) * NACC)[:, None]
    srcoff4 = (src_p[None, :] + chunk_off).reshape(4 * NS * 2, NB_HALF, BATCH)
    srcoff3 = srcoff4[:3 * NS * 2]
    srcoff2 = srcoff4[:2 * NS * 2]

    W2p = (jnp.zeros((512, 384), jnp.float32).at[:, :341].set(W2)
           .reshape(4, 128, 3, 128).transpose(0, 2, 1, 3))
    W3p = (jnp.zeros((384, 256), jnp.float32).at[:341, :227].set(W3)
           .reshape(3, 128, 2, 128).transpose(0, 2, 1, 3))
    Wcp = (jnp.zeros((256, 128), jnp.float32).at[:227, :40].set(Wc)
           .reshape(2, 128, 1, 128).transpose(0, 2, 1, 3))
    b1r = b1.reshape(4, 1, 128)
    b2p = jnp.concatenate([b2, jnp.zeros((43,), jnp.float32)]).reshape(3, 1, 128)
    b3p = jnp.concatenate([b3, jnp.zeros((29,), jnp.float32)]).reshape(2, 1, 128)
    bcp = jnp.concatenate([bc, jnp.zeros((88,), jnp.float32)]).reshape(1, 1, 128)
    ones_col = jnp.ones((NACC, 1), jnp.float32)

    cnt = _sc_deg(dst_deg).reshape(NC, NACC, 128)
    deg = (cnt[0, :, 0] + cnt[1, :, 0] + 1.0).reshape(NACC, 1)

    u1 = _tc1(x, W1, deg)
    agg1 = _sc_agg(u1.reshape(4 * NACC, 128), srcoff4, dst_agg,
                   [(0, 0, 2, 0), (1, 0, 2, 1)],
                   [(2, 0, 2, 2), (3, 0, 2, 3)], 4, 128)
    u2 = _tc2(agg1, u1, b1r, deg, W2p, deg, jnp.zeros((3, 1, 128), jnp.float32),
              nci=4, fci=128, nco=3, fco=128)
    agg2p = _sc_agg(u2.reshape(3 * NACC, 128), srcoff3, dst_agg,
                    [(0, 0, 2, 0), (2, 0, 1, 2)],
                    [(1, 0, 2, 1), (2, 1, 2, 3)], 4, 128)
    agg2 = jnp.concatenate([agg2p[:2], (agg2p[2] + agg2p[3])[None]], axis=0)
    u3 = _tc2(agg2, u2, b2p, deg, W3p, deg, jnp.zeros((2, 1, 128), jnp.float32),
              nci=3, fci=128, nco=2, fco=128)
    agg3 = _sc_agg(u3.reshape(2 * NACC, 128), srcoff2, dst_agg,
                   [(0, 0, 2, 0)], [(1, 0, 2, 1)], 2, 128)
    outp = _tc2(agg3, u3, b3p, deg, Wcp, ones_col, bcp,
                nci=2, fci=128, nco=1, fco=128)
    return outp[0, :N_NODES, :40]


# speedup vs baseline: 1.0972x; 1.0972x over previous
import functools

import jax
import jax.numpy as jnp
from jax import lax
from jax.experimental import pallas as pl
from jax.experimental.pallas import tpu as pltpu
from jax.experimental.pallas import tpu_sc as plsc

N_NODES = 10000
N_EDGES = 160000
NC = 2
NS = 16
BATCH = 128
EPAD = 163840
NB_AGG = EPAD // (NS * BATCH)
NB_HALF = NB_AGG // 2
NB_DEG = EPAD // (NC * NS * BATCH)
RPT = 632
NACC = NS * RPT
DUMP_ROW = N_NODES
RB = 1000
RB2 = 1000


def _zero_vmem_rows(ref, nrows, ncols):
    zero16 = jnp.zeros((16,), jnp.float32)

    def row(i, carry):
        for k in range(ncols // 16):
            ref[i, pl.ds(k * 16, 16)] = zero16
        return carry

    lax.fori_loop(0, nrows, row, 0)


def _deg_body(dst_hbm, out_hbm, didx, onesb, zbuf, acc):
    c = lax.axis_index("c")
    s = lax.axis_index("s")
    w = c * NS + s

    one16 = jnp.ones((16,), jnp.float32)

    def fill(i, carry):
        for k in range(128 // 16):
            onesb[i, pl.ds(k * 16, 16)] = one16
        return carry

    lax.fori_loop(0, BATCH, fill, 0)
    _zero_vmem_rows(zbuf, BATCH, 128)
    for t in range(-(-RPT // BATCH)):
        rows = min(BATCH, RPT - t * BATCH)
        pltpu.sync_copy(zbuf.at[pl.ds(0, rows)],
                        acc.at[pl.ds(s * RPT + t * BATCH, rows)])
    plsc.subcore_barrier()

    pltpu.sync_copy(dst_hbm.at[w], didx)

    def ebody(j, carry):
        pltpu.sync_copy(onesb, acc.at[didx.at[j]], add=True)
        return carry

    lax.fori_loop(0, NB_DEG, ebody, 0)
    plsc.subcore_barrier()
    for t in range(-(-RPT // BATCH)):
        rows = min(BATCH, RPT - t * BATCH)
        pltpu.sync_copy(acc.at[pl.ds(s * RPT + t * BATCH, rows)],
                        zbuf.at[pl.ds(0, rows)])
        pltpu.sync_copy(zbuf.at[pl.ds(0, rows)],
                        out_hbm.at[pl.ds(c * NACC + s * RPT + t * BATCH, rows)])


def _sc_deg(dst_deg):
    mesh = plsc.VectorSubcoreMesh(core_axis_name="c", subcore_axis_name="s")
    return pl.kernel(
        _deg_body,
        out_type=jax.ShapeDtypeStruct((NC * NACC, 128), jnp.float32),
        mesh=mesh,
        scratch_types=[
            pltpu.VMEM((NB_DEG, BATCH), jnp.int32),
            pltpu.VMEM((BATCH, 128), jnp.float32),
            pltpu.VMEM((BATCH, 128), jnp.float32),
            pltpu.VMEM_SHARED((NACC, 128), jnp.float32),
        ],
    )(dst_deg)


def _agg_run_slot(ck, h0, h1, slot, fc, u_hbm, srcoff_hbm, dst_hbm, out_hbm,
                  sidx, didx, gbufa, gbufb, sema, semb, acc, s):
    _zero_vmem_rows(gbufa, BATCH, fc)
    zcps = []
    for t in range(-(-RPT // BATCH)):
        rows = min(BATCH, RPT - t * BATCH)
        zcps.append(pltpu.async_copy(
            gbufa.at[pl.ds(0, rows)],
            acc.at[pl.ds(s * RPT + t * BATCH, rows)], sema))
    for cp in zcps:
        cp.wait()
    plsc.subcore_barrier()

    for h in range(h0, h1):
        cps = [pltpu.async_copy(srcoff_hbm.at[(ck * NS + s) * 2 + h], sidx, sema),
               pltpu.async_copy(dst_hbm.at[s * 2 + h], didx, semb)]
        for cp in cps:
            cp.wait()

        pltpu.async_copy(u_hbm.at[sidx.at[0]], gbufa, sema)

        def pair(i, carry):
            j0 = 2 * i
            pltpu.async_copy(u_hbm.at[sidx.at[j0 + 1]], gbufb, semb)
            pltpu.make_async_copy(u_hbm.at[sidx.at[j0]], gbufa, sema).wait()
            pltpu.sync_copy(gbufa, acc.at[didx.at[j0]], add=True)

            @pl.when(j0 + 2 < NB_HALF)
            def _():
                pltpu.async_copy(u_hbm.at[sidx.at[j0 + 2]], gbufa, sema)

            pltpu.make_async_copy(u_hbm.at[sidx.at[j0 + 1]], gbufb, semb).wait()
            pltpu.sync_copy(gbufb, acc.at[didx.at[j0 + 1]], add=True)
            return carry

        lax.fori_loop(0, NB_HALF // 2, pair, 0)
    plsc.subcore_barrier()
    stages = [gbufa, gbufb]
    wcps = {}
    for t in range(-(-RPT // BATCH)):
        rows = min(BATCH, RPT - t * BATCH)
        stg = stages[t % 2]
        if t >= 2:
            wcps[t - 2].wait()
        pltpu.async_copy(acc.at[pl.ds(s * RPT + t * BATCH, rows)],
                         stg.at[pl.ds(0, rows)], sema).wait()
        wcps[t] = pltpu.async_copy(
            stg.at[pl.ds(0, rows)],
            out_hbm.at[pl.ds(slot * NACC + s * RPT + t * BATCH, rows)], semb)
    for t in sorted(wcps)[-2:]:
        wcps[t].wait()


def _agg_body(sched0, sched1, fc, u_hbm, srcoff_hbm, dst_hbm, out_hbm,
              sidx, didx, gbufa, gbufb, sema, semb, acc):
    c = lax.axis_index("c")
    s = lax.axis_index("s")
    args = (fc, u_hbm, srcoff_hbm, dst_hbm, out_hbm,
            sidx, didx, gbufa, gbufb, sema, semb, acc, s)

    @pl.when(c == 0)
    def _core0():
        for ck, h0, h1, slot in sched0:
            _agg_run_slot(ck, h0, h1, slot, *args)

    @pl.when(c == 1)
    def _core1():
        for ck, h0, h1, slot in sched1:
            _agg_run_slot(ck, h0, h1, slot, *args)


def _sc_agg(u_flat, srcoff, dst_agg, sched0, sched1, nslots, fc):
    mesh = plsc.VectorSubcoreMesh(core_axis_name="c", subcore_axis_name="s")
    out = pl.kernel(
        functools.partial(_agg_body, sched0, sched1, fc),
        out_type=jax.ShapeDtypeStruct((nslots * NACC, fc), jnp.float32),
        mesh=mesh,
        scratch_types=[
            pltpu.VMEM((NB_HALF, BATCH), jnp.int32),
            pltpu.VMEM((NB_HALF, BATCH), jnp.int32),
            pltpu.VMEM((BATCH, fc), jnp.float32),
            pltpu.VMEM((BATCH, fc), jnp.float32),
            pltpu.SemaphoreType.DMA,
            pltpu.SemaphoreType.DMA,
            pltpu.VMEM_SHARED((NACC, fc), jnp.float32),
        ],
    )(u_flat, srcoff, dst_agg)
    return out.reshape(nslots, NACC, fc)


def _tc1_body(x_ref, w_ref, deg_ref, out_ref):
    y = jnp.dot(x_ref[...], w_ref[...], preferred_element_type=jnp.float32)
    out_ref[0] = y * lax.rsqrt(deg_ref[...])


def _tc1(x, W1, deg):
    return pl.pallas_call(
        _tc1_body,
        grid=(4, N_NODES // RB),
        in_specs=[
            pl.BlockSpec((RB, 256), lambda co, r: (r, 0)),
            pl.BlockSpec((256, 128), lambda co, r: (0, co)),
            pl.BlockSpec((RB, 1), lambda co, r: (r, 0)),
        ],
        out_specs=pl.BlockSpec((1, RB, 128), lambda co, r: (co, r, 0)),
        out_shape=jax.ShapeDtypeStruct((4, NACC, 128), jnp.float32),
        compiler_params=pltpu.CompilerParams(
            dimension_semantics=("parallel", "parallel")),
    )(x, W1, deg)


def _tc2_body(nci, agg_ref, u_ref, b_ref, deg_ref, w_ref, dego_ref, bo_ref, out_ref):
    ci = pl.program_id(2)
    h = (agg_ref[0] + u_ref[0]) * lax.rsqrt(deg_ref[...]) + b_ref[0]
    h = jnp.where(h >= 0, h, 0.01 * h)
    part = jnp.dot(h, w_ref[0, 0], preferred_element_type=jnp.float32)

    @pl.when(ci == 0)
    def _init():
        out_ref[0] = part

    @pl.when(ci > 0)
    def _acc():
        out_ref[0] += part

    @pl.when(ci == nci - 1)
    def _fin():
        out_ref[0] = out_ref[0] * lax.rsqrt(dego_ref[...]) + bo_ref[0]


def _tc2(agg, u, bl, deg, W, dego, bo, nci, fci, nco, fco):
    return pl.pallas_call(
        functools.partial(_tc2_body, nci),
        grid=(nco, N_NODES // RB2, nci),
        in_specs=[
            pl.BlockSpec((1, RB2, fci), lambda co, r, ci: (ci, r, 0)),
            pl.BlockSpec((1, RB2, fci), lambda co, r, ci: (ci, r, 0)),
            pl.BlockSpec((1, 1, fci), lambda co, r, ci: (ci, 0, 0)),
            pl.BlockSpec((RB2, 1), lambda co, r, ci: (r, 0)),
            pl.BlockSpec((1, 1, fci, fco), lambda co, r, ci: (ci, co, 0, 0)),
            pl.BlockSpec((RB2, 1), lambda co, r, ci: (r, 0)),
            pl.BlockSpec((1, 1, fco), lambda co, r, ci: (co, 0, 0)),
        ],
        out_specs=pl.BlockSpec((1, RB2, fco), lambda co, r, ci: (co, r, 0)),
        out_shape=jax.ShapeDtypeStruct((nco, NACC, fco), jnp.float32),
        compiler_params=pltpu.CompilerParams(
            dimension_semantics=("parallel", "parallel", "arbitrary")),
    )(agg, u, bl, deg, W, dego, bo)


def kernel(x, edge_index, W1, b1, W2, b2, W3, b3, Wc, bc):
    src = edge_index[0].astype(jnp.int32)
    dst = edge_index[1].astype(jnp.int32)
    pad = EPAD - N_EDGES
    src_p = jnp.concatenate([src, jnp.zeros((pad,), jnp.int32)])
    dst_p = jnp.concatenate([dst, jnp.full((pad,), DUMP_ROW, jnp.int32)])
    dst_agg = dst_p.reshape(NS * 2, NB_HALF, BATCH)
    dst_deg = dst_p.reshape(NC * NS, NB_DEG, BATCH)
    chunk_off = (jnp.arange(4, dtype=jnp.int32) * NACC)[:, None]
    srcoff4 = (src_p[None, :] + chunk_off).reshape(4 * NS * 2, NB_HALF, BATCH)
    srcoff3 = srcoff4[:3 * NS * 2]
    srcoff2 = srcoff4[:2 * NS * 2]

    W2p = (jnp.zeros((512, 384), jnp.float32).at[:, :341].set(W2)
           .reshape(4, 128, 3, 128).transpose(0, 2, 1, 3))
    W3p = (jnp.zeros((384, 256), jnp.float32).at[:341, :227].set(W3)
           .reshape(3, 128, 2, 128).transpose(0, 2, 1, 3))
    Wcp = (jnp.zeros((256, 128), jnp.float32).at[:227, :40].set(Wc)
           .reshape(2, 128, 1, 128).transpose(0, 2, 1, 3))
    b1r = b1.reshape(4, 1, 128)
    b2p = jnp.concatenate([b2, jnp.zeros((43,), jnp.float32)]).reshape(3, 1, 128)
    b3p = jnp.concatenate([b3, jnp.zeros((29,), jnp.float32)]).reshape(2, 1, 128)
    bcp = jnp.concatenate([bc, jnp.zeros((88,), jnp.float32)]).reshape(1, 1, 128)
    ones_col = jnp.ones((NACC, 1), jnp.float32)

    cnt = _sc_deg(dst_deg).reshape(NC, NACC, 128)
    deg = (cnt[0, :, 0] + cnt[1, :, 0] + 1.0).reshape(NACC, 1)

    u1 = _tc1(x, W1, deg)
    agg1 = _sc_agg(u1.reshape(4 * NACC, 128), srcoff4, dst_agg,
                   [(0, 0, 2, 0), (1, 0, 2, 1)],
                   [(2, 0, 2, 2), (3, 0, 2, 3)], 4, 128)
    u2 = _tc2(agg1, u1, b1r, deg, W2p, deg, jnp.zeros((3, 1, 128), jnp.float32),
              nci=4, fci=128, nco=3, fco=128)
    agg2p = _sc_agg(u2.reshape(3 * NACC, 128), srcoff3, dst_agg,
                    [(0, 0, 2, 0), (2, 0, 1, 2)],
                    [(1, 0, 2, 1), (2, 1, 2, 3)], 4, 128)
    agg2 = jnp.concatenate([agg2p[:2], (agg2p[2] + agg2p[3])[None]], axis=0)
    u3 = _tc2(agg2, u2, b2p, deg, W3p, deg, jnp.zeros((2, 1, 128), jnp.float32),
              nci=3, fci=128, nco=2, fco=128)
    agg3 = _sc_agg(u3.reshape(2 * NACC, 128), srcoff2, dst_agg,
                   [(0, 0, 2, 0)], [(1, 0, 2, 1)], 2, 128)
    outp = _tc2(agg3, u3, b3p, deg, Wcp, ones_col, bcp,
                nci=2, fci=128, nco=1, fco=128)
    return outp[0, :N_NODES, :40]
